# baseline (speedup 1.0000x reference)
import jax
from jax import lax
import jax.numpy as jnp
from jax.experimental import pallas as pl
from jax.experimental.pallas import tpu as pltpu
from jax.experimental.pallas import tpu_sc as plsc

_C = 64
_T = 16384
_B = 8
_OUT_T = 32768
_L = 16
_W = 2048
_NSUB = 17

_OFFSETS = (1220, 18, 1207, 3265, 653, 3435, 2433, 2343)

_MESH = plsc.VectorSubcoreMesh(core_axis_name="c", subcore_axis_name="s")


def _sel_offset(b):
    off = jnp.int32(_OFFSETS[0])
    for i in range(1, _B):
        off = jnp.where(b == i, jnp.int32(_OFFSETS[i]), off)
    return off


def _jitter_sc(x_hbm, out_hbm, xxc0, xxc1, rot0, rot1, rot2, rot3,
               semi0, semi1, semo0, semo1, semo2, semo3):
    wid = lax.axis_index("s") * 2 + lax.axis_index("c")
    o8 = pl.multiple_of((wid >> 2) * 8, 8)
    q = wid & 3
    lanes = lax.iota(jnp.int32, _L)
    xxcs, semis = (xxc0, xxc1), (semi0, semi1)
    rots, semos = (rot0, rot1, rot2, rot3), (semo0, semo1, semo2, semo3)

    def issue_in(batch, j2, slot):
        off = _sel_offset(batch)
        base = ((off >> 7) << 7) + (4 * q + 2 * j2) * 1024
        xxc, sem = xxcs[slot], semis[slot]

        def body(i, carry):
            src_col = pl.multiple_of((base + 128 * i) & (_T - 1), 128)
            dst_col = pl.multiple_of(128 * i, 128)
            pltpu.async_copy(
                x_hbm.at[0, pl.ds(o8, 8), pl.ds(src_col, 128)],
                xxc.at[:, pl.ds(dst_col, 128)],
                sem)
            return carry

        lax.fori_loop(0, _NSUB, body, 0)

    def wait_in(slot):
        xxc, sem = xxcs[slot], semis[slot]
        for _ in range(_NSUB):
            pltpu.make_async_copy(
                x_hbm.at[0, pl.ds(o8, 8), pl.ds(0, 128)],
                xxc.at[:, pl.ds(0, 128)],
                sem).wait()

    def drain_out(slot):
        for _ in range(2):
            pltpu.make_async_copy(
                rots[slot],
                out_hbm.at[0, pl.ds(o8, 8), pl.ds(0, _W)],
                semos[slot]).wait()

    def produce(batch, j2, in_slot, out_slot):
        off = _sel_offset(batch)
        r = off & 127
        rl = r + lanes
        xxc, rot = xxcs[in_slot], rots[out_slot]

        def body(j):
            for u in range(8):
                col_l = (j + u * _L) + rl
                col_s = (j + u * _L) + lanes
                for c in range(8):
                    rowv = jnp.full((_L,), c, jnp.int32)
                    vals = plsc.load_gather(xxc, [rowv, col_l])
                    plsc.store_scatter(rot, [rowv, col_s], vals)

        plsc.parallel_loop(0, _W, 128)(body)
        t0 = pl.multiple_of((4 * q + 2 * j2) * 1024, _W)
        for half in range(2):
            dst_col = pl.multiple_of(t0 + half * _T, _W)
            pltpu.async_copy(
                rot, out_hbm.at[batch, pl.ds(o8, 8), pl.ds(dst_col, _W)],
                semos[out_slot])

    issue_in(jnp.int32(0), 0, 0)

    def step(s, carry):
        for e in range(2):
            batch = 2 * s + e
            for j2 in range(2):
                ci = e * 2 + j2
                wait_in(ci % 2)
                if ci + 1 < 4:
                    nb, nj = batch + (j2 == 1), (ci + 1) % 2
                    issue_in(nb, nj, (ci + 1) % 2)
                else:
                    @pl.when(s < _B // 2 - 1)
                    def _():
                        issue_in(2 * s + 2, 0, 0)

                @pl.when(s > 0)
                def _():
                    drain_out(ci)

                produce(batch, j2, ci % 2, ci)
        return carry

    lax.fori_loop(0, _B // 2, step, 0)
    for sl in range(4):
        drain_out(sl)


_CALL = pl.kernel(
    _jitter_sc,
    out_type=jax.ShapeDtypeStruct((_B, _C, _OUT_T), jnp.float32),
    mesh=_MESH,
    scratch_types=[
        pltpu.VMEM((8, _NSUB * 128), jnp.float32),
        pltpu.VMEM((8, _NSUB * 128), jnp.float32),
        pltpu.VMEM((8, _W), jnp.float32),
        pltpu.VMEM((8, _W), jnp.float32),
        pltpu.VMEM((8, _W), jnp.float32),
        pltpu.VMEM((8, _W), jnp.float32),
        pltpu.SemaphoreType.DMA,
        pltpu.SemaphoreType.DMA,
        pltpu.SemaphoreType.DMA,
        pltpu.SemaphoreType.DMA,
        pltpu.SemaphoreType.DMA,
        pltpu.SemaphoreType.DMA,
    ],
    compiler_params=pltpu.CompilerParams(needs_layout_passes=False),
)


def kernel(x):
    return _CALL(x)


# speedup vs baseline: 1.1228x; 1.1228x over previous
import jax
from jax import lax
import jax.numpy as jnp
from jax.experimental import pallas as pl
from jax.experimental.pallas import tpu as pltpu
from jax.experimental.pallas import tpu_sc as plsc

_C = 64
_T = 16384
_B = 8
_OUT_T = 32768
_L = 16
_W = 2048
_NSUB = 17

_OFFSETS = (1220, 18, 1207, 3265, 653, 3435, 2433, 2343)

_MESH = plsc.VectorSubcoreMesh(core_axis_name="c", subcore_axis_name="s")


def _sel_offset(b):
    off = jnp.int32(_OFFSETS[0])
    for i in range(1, _B):
        off = jnp.where(b == i, jnp.int32(_OFFSETS[i]), off)
    return off


def _jitter_sc(x_hbm, out_hbm, xxc0, xxc1, rot0, rot1,
               semi0, semi1, semo0, semo1):
    wid = lax.axis_index("s") * 2 + lax.axis_index("c")
    o8 = pl.multiple_of((wid >> 2) * 8, 8)
    q = wid & 3
    lanes = lax.iota(jnp.int32, _L)
    xxcs, semis = (xxc0, xxc1), (semi0, semi1)
    rots, semos = (rot0, rot1), (semo0, semo1)

    def issue_in(batch, j2, slot):
        off = _sel_offset(batch)
        base = ((off >> 7) << 7) + (4 * q + 2 * j2) * 1024
        xxc, sem = xxcs[slot], semis[slot]
        fits_lo = base + _NSUB * 128 <= _T
        fits_hi = base >= _T

        @pl.when(fits_lo)
        def _():
            pltpu.async_copy(
                x_hbm.at[0, pl.ds(o8, 8),
                         pl.ds(pl.multiple_of(base, 128), _NSUB * 128)],
                xxc, sem)

        @pl.when(fits_hi)
        def _():
            pltpu.async_copy(
                x_hbm.at[0, pl.ds(o8, 8),
                         pl.ds(pl.multiple_of(base - _T, 128), _NSUB * 128)],
                xxc, sem)

        @pl.when(jnp.logical_not(jnp.logical_or(fits_lo, fits_hi)))
        def _():
            def body(i, carry):
                src_col = pl.multiple_of((base + 128 * i) & (_T - 1), 128)
                dst_col = pl.multiple_of(128 * i, 128)
                pltpu.async_copy(
                    x_hbm.at[0, pl.ds(o8, 8), pl.ds(src_col, 128)],
                    xxc.at[:, pl.ds(dst_col, 128)],
                    sem)
                return carry

            lax.fori_loop(0, _NSUB, body, 0)

    def wait_in(slot):
        pltpu.make_async_copy(
            x_hbm.at[0, pl.ds(o8, 8), pl.ds(0, _NSUB * 128)],
            xxcs[slot], semis[slot]).wait()

    def drain_out(slot):
        for _ in range(2):
            pltpu.make_async_copy(
                rots[slot],
                out_hbm.at[0, pl.ds(o8, 8), pl.ds(0, _W)],
                semos[slot]).wait()

    def produce(batch, j2, slot):
        off = _sel_offset(batch)
        r = off & 127
        rl = r + lanes
        xxc, rot = xxcs[slot], rots[slot]

        def body(j):
            for u in range(8):
                col = j + u * _L
                for c in range(8):
                    rowv = jnp.full((_L,), c, jnp.int32)
                    vals = plsc.load_gather(xxc, [rowv, col + rl])
                    plsc.store_scatter(rot, [rowv, col + lanes], vals)

        plsc.parallel_loop(0, _W, 128)(body)
        t0 = pl.multiple_of((4 * q + 2 * j2) * 1024, _W)
        for half in range(2):
            dst_col = pl.multiple_of(t0 + half * _T, _W)
            pltpu.async_copy(
                rot, out_hbm.at[batch, pl.ds(o8, 8), pl.ds(dst_col, _W)],
                semos[slot])

    issue_in(jnp.int32(0), 0, 0)

    def step(m, carry):
        wait_in(0)
        issue_in(m, 1, 1)

        @pl.when(m > 0)
        def _():
            drain_out(0)

        produce(m, 0, 0)
        wait_in(1)

        @pl.when(m < _B - 1)
        def _():
            issue_in(m + 1, 0, 0)

        @pl.when(m > 0)
        def _():
            drain_out(1)

        produce(m, 1, 1)
        return carry

    lax.fori_loop(0, _B, step, 0)
    drain_out(0)
    drain_out(1)


_CALL = pl.kernel(
    _jitter_sc,
    out_type=jax.ShapeDtypeStruct((_B, _C, _OUT_T), jnp.float32),
    mesh=_MESH,
    scratch_types=[
        pltpu.VMEM((8, _NSUB * 128), jnp.float32),
        pltpu.VMEM((8, _NSUB * 128), jnp.float32),
        pltpu.VMEM((8, _W), jnp.float32),
        pltpu.VMEM((8, _W), jnp.float32),
        pltpu.SemaphoreType.DMA,
        pltpu.SemaphoreType.DMA,
        pltpu.SemaphoreType.DMA,
        pltpu.SemaphoreType.DMA,
    ],
    compiler_params=pltpu.CompilerParams(needs_layout_passes=False),
)


def kernel(x):
    return _CALL(x)


# speedup vs baseline: 1.2771x; 1.1374x over previous
import jax
from jax import lax
import jax.numpy as jnp
from jax.experimental import pallas as pl
from jax.experimental.pallas import tpu as pltpu
from jax.experimental.pallas import tpu_sc as plsc

_C = 64
_T = 16384
_B = 8
_OUT_T = 32768
_L = 16
_W = 4096
_XW = 7552

_OFFSETS = (1220, 18, 1207, 3265, 653, 3435, 2433, 2343)

_MESH = plsc.VectorSubcoreMesh(core_axis_name="c", subcore_axis_name="s")


def _sel_offset(b):
    off = jnp.int32(_OFFSETS[0])
    for i in range(1, _B):
        off = jnp.where(b == i, jnp.int32(_OFFSETS[i]), off)
    return off


def _jitter_sc(x_hbm, out_hbm, xbig, rot0, rot1, semi, semo0, semo1):
    wid = lax.axis_index("s") * 2 + lax.axis_index("c")
    o8 = pl.multiple_of((wid >> 2) * 8, 8)
    q = wid & 3
    lanes = lax.iota(jnp.int32, _L)
    rots, semos = (rot0, rot1), (semo0, semo1)

    @pl.when(q < 3)
    def _():
        pltpu.async_copy(
            x_hbm.at[0, pl.ds(o8, 8),
                     pl.ds(pl.multiple_of(q * 4096, 128), _XW)],
            xbig, semi)

    @pl.when(q == 3)
    def _():
        pltpu.async_copy(
            x_hbm.at[0, pl.ds(o8, 8), pl.ds(12288, 4096)],
            xbig.at[:, pl.ds(0, 4096)], semi)
        pltpu.async_copy(
            x_hbm.at[0, pl.ds(o8, 8), pl.ds(0, _XW - 4096)],
            xbig.at[:, pl.ds(4096, _XW - 4096)], semi)

    @pl.when(q < 3)
    def _():
        pltpu.make_async_copy(
            x_hbm.at[0, pl.ds(o8, 8), pl.ds(0, _XW)], xbig, semi).wait()

    @pl.when(q == 3)
    def _():
        pltpu.make_async_copy(
            x_hbm.at[0, pl.ds(o8, 8), pl.ds(0, 4096)],
            xbig.at[:, pl.ds(0, 4096)], semi).wait()
        pltpu.make_async_copy(
            x_hbm.at[0, pl.ds(o8, 8), pl.ds(0, _XW - 4096)],
            xbig.at[:, pl.ds(4096, _XW - 4096)], semi).wait()

    def drain_out(slot):
        for _ in range(2):
            pltpu.make_async_copy(
                rots[slot],
                out_hbm.at[0, pl.ds(o8, 8), pl.ds(0, _W)],
                semos[slot]).wait()

    def produce(batch, slot):
        off = _sel_offset(batch)
        rl = off + lanes
        rot = rots[slot]

        def body(j):
            for u in range(8):
                col_l = (j + u * _L) + rl
                col_s = (j + u * _L) + lanes
                for c in range(8):
                    rowv = jnp.full((_L,), c, jnp.int32)
                    vals = plsc.load_gather(xbig, [rowv, col_l])
                    plsc.store_scatter(rot, [rowv, col_s], vals)

        plsc.parallel_loop(0, _W, 128)(body)
        t0 = pl.multiple_of(q * 4096, 128)
        for half in range(2):
            dst_col = pl.multiple_of(t0 + half * _T, 128)
            pltpu.async_copy(
                rot, out_hbm.at[batch, pl.ds(o8, 8), pl.ds(dst_col, _W)],
                semos[slot])

    def step(s, carry):
        for e in range(2):
            batch = 2 * s + e

            @pl.when(s > 0)
            def _():
                drain_out(e)

            produce(batch, e)
        return carry

    lax.fori_loop(0, _B // 2, step, 0)
    drain_out(0)
    drain_out(1)


_CALL = pl.kernel(
    _jitter_sc,
    out_type=jax.ShapeDtypeStruct((_B, _C, _OUT_T), jnp.float32),
    mesh=_MESH,
    scratch_types=[
        pltpu.VMEM((8, _XW), jnp.float32),
        pltpu.VMEM((8, _W), jnp.float32),
        pltpu.VMEM((8, _W), jnp.float32),
        pltpu.SemaphoreType.DMA,
        pltpu.SemaphoreType.DMA,
        pltpu.SemaphoreType.DMA,
    ],
    compiler_params=pltpu.CompilerParams(needs_layout_passes=False),
)


def kernel(x):
    return _CALL(x)


# speedup vs baseline: 1.5890x; 1.2443x over previous
import jax
from jax import lax
import jax.numpy as jnp
from jax.experimental import pallas as pl
from jax.experimental.pallas import tpu as pltpu
from jax.experimental.pallas import tpu_sc as plsc

_C = 64
_T = 16384
_B = 8
_OUT_T = 32768
_L = 16
_W = 4096
_XW = 7552

_OFFSETS = (1220, 18, 1207, 3265, 653, 3435, 2433, 2343)

_MESH = plsc.VectorSubcoreMesh(core_axis_name="c", subcore_axis_name="s")


def _sel_offset(b):
    off = jnp.int32(_OFFSETS[0])
    for i in range(1, _B):
        off = jnp.where(b == i, jnp.int32(_OFFSETS[i]), off)
    return off


def _jitter_sc(x_hbm, out_hbm, xbig, rot0, rot1, semi, semo0, semo1):
    wid = lax.axis_index("s") * 2 + lax.axis_index("c")
    o8 = pl.multiple_of((wid >> 2) * 8, 8)
    q = wid & 3
    lanes = lax.iota(jnp.int32, _L)
    rots, semos = (rot0, rot1), (semo0, semo1)

    @pl.when(q < 3)
    def _():
        pltpu.async_copy(
            x_hbm.at[0, pl.ds(o8, 8),
                     pl.ds(pl.multiple_of(q * 4096, 128), _XW)],
            xbig, semi)

    @pl.when(q == 3)
    def _():
        pltpu.async_copy(
            x_hbm.at[0, pl.ds(o8, 8), pl.ds(12288, 4096)],
            xbig.at[:, pl.ds(0, 4096)], semi)
        pltpu.async_copy(
            x_hbm.at[0, pl.ds(o8, 8), pl.ds(0, _XW - 4096)],
            xbig.at[:, pl.ds(4096, _XW - 4096)], semi)

    @pl.when(q < 3)
    def _():
        pltpu.make_async_copy(
            x_hbm.at[0, pl.ds(o8, 8), pl.ds(0, _XW)], xbig, semi).wait()

    @pl.when(q == 3)
    def _():
        pltpu.make_async_copy(
            x_hbm.at[0, pl.ds(o8, 8), pl.ds(0, 4096)],
            xbig.at[:, pl.ds(0, 4096)], semi).wait()
        pltpu.make_async_copy(
            x_hbm.at[0, pl.ds(o8, 8), pl.ds(0, _XW - 4096)],
            xbig.at[:, pl.ds(4096, _XW - 4096)], semi).wait()

    def drain_out(slot):
        for _ in range(2):
            pltpu.make_async_copy(
                rots[slot],
                out_hbm.at[0, pl.ds(o8, 8), pl.ds(0, _W)],
                semos[slot]).wait()

    def produce(batch, slot):
        off = _sel_offset(batch)
        rl = off + lanes
        rot = rots[slot]

        def body(j):
            for u in range(4):
                col_l = (j + u * _L) + rl
                col_s = (j + u * _L) + lanes
                for c in range(8):
                    rowv = jnp.full((_L,), c, jnp.int32)
                    vals = plsc.load_gather(xbig, [rowv, col_l])
                    plsc.store_scatter(rot, [rowv, col_s], vals)

        plsc.parallel_loop(0, _W, 64)(body)
        t0 = pl.multiple_of(q * 4096, 128)
        for half in range(2):
            dst_col = pl.multiple_of(t0 + half * _T, 128)
            pltpu.async_copy(
                rot, out_hbm.at[batch, pl.ds(o8, 8), pl.ds(dst_col, _W)],
                semos[slot])

    def step(s, carry):
        for e in range(2):
            batch = 2 * s + e

            @pl.when(s > 0)
            def _():
                drain_out(e)

            produce(batch, e)
        return carry

    lax.fori_loop(0, _B // 2, step, 0)
    drain_out(0)
    drain_out(1)


_CALL = pl.kernel(
    _jitter_sc,
    out_type=jax.ShapeDtypeStruct((_B, _C, _OUT_T), jnp.float32),
    mesh=_MESH,
    scratch_types=[
        pltpu.VMEM((8, _XW), jnp.float32),
        pltpu.VMEM((8, _W), jnp.float32),
        pltpu.VMEM((8, _W), jnp.float32),
        pltpu.SemaphoreType.DMA,
        pltpu.SemaphoreType.DMA,
        pltpu.SemaphoreType.DMA,
    ],
    compiler_params=pltpu.CompilerParams(needs_layout_passes=False),
)


def kernel(x):
    return _CALL(x)


# speedup vs baseline: 1.6160x; 1.0170x over previous
import jax
from jax import lax
import jax.numpy as jnp
from jax.experimental import pallas as pl
from jax.experimental.pallas import tpu as pltpu
from jax.experimental.pallas import tpu_sc as plsc

_C = 64
_T = 16384
_B = 8
_OUT_T = 32768
_L = 16
_W = 4096
_XW = 7552

_OFFSETS = (1220, 18, 1207, 3265, 653, 3435, 2433, 2343)

_MESH = plsc.VectorSubcoreMesh(core_axis_name="c", subcore_axis_name="s")


def _sel_offset(b):
    off = jnp.int32(_OFFSETS[0])
    for i in range(1, _B):
        off = jnp.where(b == i, jnp.int32(_OFFSETS[i]), off)
    return off


def _jitter_sc(x_hbm, out_hbm, xbig, rot0, rot1, semi, semo0, semo1):
    wid = lax.axis_index("s") * 2 + lax.axis_index("c")
    o8 = pl.multiple_of((wid >> 2) * 8, 8)
    q = wid & 3
    lanes = lax.iota(jnp.int32, _L)
    rots, semos = (rot0, rot1), (semo0, semo1)

    @pl.when(q < 3)
    def _():
        pltpu.async_copy(
            x_hbm.at[0, pl.ds(o8, 8),
                     pl.ds(pl.multiple_of(q * 4096, 128), _XW)],
            xbig, semi)

    @pl.when(q == 3)
    def _():
        pltpu.async_copy(
            x_hbm.at[0, pl.ds(o8, 8), pl.ds(12288, 4096)],
            xbig.at[:, pl.ds(0, 4096)], semi)
        pltpu.async_copy(
            x_hbm.at[0, pl.ds(o8, 8), pl.ds(0, _XW - 4096)],
            xbig.at[:, pl.ds(4096, _XW - 4096)], semi)

    @pl.when(q < 3)
    def _():
        pltpu.make_async_copy(
            x_hbm.at[0, pl.ds(o8, 8), pl.ds(0, _XW)], xbig, semi).wait()

    @pl.when(q == 3)
    def _():
        pltpu.make_async_copy(
            x_hbm.at[0, pl.ds(o8, 8), pl.ds(0, 4096)],
            xbig.at[:, pl.ds(0, 4096)], semi).wait()
        pltpu.make_async_copy(
            x_hbm.at[0, pl.ds(o8, 8), pl.ds(0, _XW - 4096)],
            xbig.at[:, pl.ds(4096, _XW - 4096)], semi).wait()

    def drain_out(slot):
        for _ in range(2):
            pltpu.make_async_copy(
                rots[slot],
                out_hbm.at[0, pl.ds(o8, 8), pl.ds(0, _W)],
                semos[slot]).wait()

    def produce(batch, slot):
        off = _sel_offset(batch)
        rl = off + lanes
        rot = rots[slot]

        def body(j):
            for u in range(2):
                col_l = (j + u * _L) + rl
                col_s = (j + u * _L) + lanes
                for c in range(8):
                    rowv = jnp.full((_L,), c, jnp.int32)
                    vals = plsc.load_gather(xbig, [rowv, col_l])
                    plsc.store_scatter(rot, [rowv, col_s], vals)

        plsc.parallel_loop(0, _W, 32)(body)
        t0 = pl.multiple_of(q * 4096, 128)
        for half in range(2):
            dst_col = pl.multiple_of(t0 + half * _T, 128)
            pltpu.async_copy(
                rot, out_hbm.at[batch, pl.ds(o8, 8), pl.ds(dst_col, _W)],
                semos[slot])

    def step(s, carry):
        for e in range(2):
            batch = 2 * s + e

            @pl.when(s > 0)
            def _():
                drain_out(e)

            produce(batch, e)
        return carry

    lax.fori_loop(0, _B // 2, step, 0)
    drain_out(0)
    drain_out(1)


_CALL = pl.kernel(
    _jitter_sc,
    out_type=jax.ShapeDtypeStruct((_B, _C, _OUT_T), jnp.float32),
    mesh=_MESH,
    scratch_types=[
        pltpu.VMEM((8, _XW), jnp.float32),
        pltpu.VMEM((8, _W), jnp.float32),
        pltpu.VMEM((8, _W), jnp.float32),
        pltpu.SemaphoreType.DMA,
        pltpu.SemaphoreType.DMA,
        pltpu.SemaphoreType.DMA,
    ],
    compiler_params=pltpu.CompilerParams(needs_layout_passes=False),
)


def kernel(x):
    return _CALL(x)


# speedup vs baseline: 1.6239x; 1.0049x over previous
import jax
from jax import lax
import jax.numpy as jnp
from jax.experimental import pallas as pl
from jax.experimental.pallas import tpu as pltpu
from jax.experimental.pallas import tpu_sc as plsc

_C = 64
_T = 16384
_B = 8
_OUT_T = 32768
_L = 16
_W = 4096
_XW = 7552

_OFFSETS = (1220, 18, 1207, 3265, 653, 3435, 2433, 2343)

_MESH = plsc.VectorSubcoreMesh(core_axis_name="c", subcore_axis_name="s")


def _sel_offset(b):
    off = jnp.int32(_OFFSETS[0])
    for i in range(1, _B):
        off = jnp.where(b == i, jnp.int32(_OFFSETS[i]), off)
    return off


def _jitter_sc(x_hbm, out_hbm, xbig, rot0, rot1, semi, semo0, semo1):
    wid = lax.axis_index("s") * 2 + lax.axis_index("c")
    o8 = pl.multiple_of((wid >> 2) * 8, 8)
    q = wid & 3
    lanes = lax.iota(jnp.int32, _L)
    rots, semos = (rot0, rot1), (semo0, semo1)

    @pl.when(q < 3)
    def _():
        pltpu.async_copy(
            x_hbm.at[0, pl.ds(o8, 8),
                     pl.ds(pl.multiple_of(q * 4096, 128), _XW)],
            xbig, semi)

    @pl.when(q == 3)
    def _():
        pltpu.async_copy(
            x_hbm.at[0, pl.ds(o8, 8), pl.ds(12288, 4096)],
            xbig.at[:, pl.ds(0, 4096)], semi)
        pltpu.async_copy(
            x_hbm.at[0, pl.ds(o8, 8), pl.ds(0, _XW - 4096)],
            xbig.at[:, pl.ds(4096, _XW - 4096)], semi)

    @pl.when(q < 3)
    def _():
        pltpu.make_async_copy(
            x_hbm.at[0, pl.ds(o8, 8), pl.ds(0, _XW)], xbig, semi).wait()

    @pl.when(q == 3)
    def _():
        pltpu.make_async_copy(
            x_hbm.at[0, pl.ds(o8, 8), pl.ds(0, 4096)],
            xbig.at[:, pl.ds(0, 4096)], semi).wait()
        pltpu.make_async_copy(
            x_hbm.at[0, pl.ds(o8, 8), pl.ds(0, _XW - 4096)],
            xbig.at[:, pl.ds(4096, _XW - 4096)], semi).wait()

    def drain_out(slot):
        for _ in range(2):
            pltpu.make_async_copy(
                rots[slot],
                out_hbm.at[0, pl.ds(o8, 8), pl.ds(0, _W)],
                semos[slot]).wait()

    def produce(batch, slot):
        off = _sel_offset(batch)
        rl = off + lanes
        rot = rots[slot]

        def body(j):
            for u in range(1):
                col_l = (j + u * _L) + rl
                col_s = (j + u * _L) + lanes
                for c in range(8):
                    rowv = jnp.full((_L,), c, jnp.int32)
                    vals = plsc.load_gather(xbig, [rowv, col_l])
                    plsc.store_scatter(rot, [rowv, col_s], vals)

        plsc.parallel_loop(0, _W, 16)(body)
        t0 = pl.multiple_of(q * 4096, 128)
        for half in range(2):
            dst_col = pl.multiple_of(t0 + half * _T, 128)
            pltpu.async_copy(
                rot, out_hbm.at[batch, pl.ds(o8, 8), pl.ds(dst_col, _W)],
                semos[slot])

    def step(s, carry):
        for e in range(2):
            batch = 2 * s + e

            @pl.when(s > 0)
            def _():
                drain_out(e)

            produce(batch, e)
        return carry

    lax.fori_loop(0, _B // 2, step, 0)
    drain_out(0)
    drain_out(1)


_CALL = pl.kernel(
    _jitter_sc,
    out_type=jax.ShapeDtypeStruct((_B, _C, _OUT_T), jnp.float32),
    mesh=_MESH,
    scratch_types=[
        pltpu.VMEM((8, _XW), jnp.float32),
        pltpu.VMEM((8, _W), jnp.float32),
        pltpu.VMEM((8, _W), jnp.float32),
        pltpu.SemaphoreType.DMA,
        pltpu.SemaphoreType.DMA,
        pltpu.SemaphoreType.DMA,
    ],
    compiler_params=pltpu.CompilerParams(needs_layout_passes=False),
)


def kernel(x):
    return _CALL(x)


# speedup vs baseline: 1.6391x; 1.0094x over previous
import jax
from jax import lax
import jax.numpy as jnp
from jax.experimental import pallas as pl
from jax.experimental.pallas import tpu as pltpu
from jax.experimental.pallas import tpu_sc as plsc

_C = 64
_T = 16384
_B = 8
_OUT_T = 32768
_L = 16
_W = 4096
_XW = 7552

_OFFSETS = (1220, 18, 1207, 3265, 653, 3435, 2433, 2343)

_MESH = plsc.VectorSubcoreMesh(core_axis_name="c", subcore_axis_name="s")


def _sel_offset(b):
    off = jnp.int32(_OFFSETS[0])
    for i in range(1, _B):
        off = jnp.where(b == i, jnp.int32(_OFFSETS[i]), off)
    return off


def _jitter_sc(x_hbm, out_hbm, xbig, rot0, rot1, semi, semo0, semo1):
    wid = lax.axis_index("s") * 2 + lax.axis_index("c")
    o8 = pl.multiple_of((wid >> 2) * 8, 8)
    q = wid & 3
    lanes = lax.iota(jnp.int32, _L)
    rots, semos = (rot0, rot1), (semo0, semo1)

    @pl.when(q < 3)
    def _():
        pltpu.async_copy(
            x_hbm.at[0, pl.ds(o8, 8),
                     pl.ds(pl.multiple_of(q * 4096, 128), _XW)],
            xbig, semi)

    @pl.when(q == 3)
    def _():
        pltpu.async_copy(
            x_hbm.at[0, pl.ds(o8, 8), pl.ds(12288, 4096)],
            xbig.at[:, pl.ds(0, 4096)], semi)
        pltpu.async_copy(
            x_hbm.at[0, pl.ds(o8, 8), pl.ds(0, _XW - 4096)],
            xbig.at[:, pl.ds(4096, _XW - 4096)], semi)

    @pl.when(q < 3)
    def _():
        pltpu.make_async_copy(
            x_hbm.at[0, pl.ds(o8, 8), pl.ds(0, _XW)], xbig, semi).wait()

    @pl.when(q == 3)
    def _():
        pltpu.make_async_copy(
            x_hbm.at[0, pl.ds(o8, 8), pl.ds(0, 4096)],
            xbig.at[:, pl.ds(0, 4096)], semi).wait()
        pltpu.make_async_copy(
            x_hbm.at[0, pl.ds(o8, 8), pl.ds(0, _XW - 4096)],
            xbig.at[:, pl.ds(4096, _XW - 4096)], semi).wait()

    def drain_out(slot):
        for _ in range(2):
            pltpu.make_async_copy(
                rots[slot],
                out_hbm.at[0, pl.ds(o8, 8), pl.ds(0, _W)],
                semos[slot]).wait()

    def produce(batch, slot):
        off = _sel_offset(batch)
        rl = off + lanes
        rot = rots[slot]

        def body(j):
            for u in range(1):
                col_l = (j + u * _L) + rl
                col_s = (j + u * _L) + lanes
                for c in range(8):
                    rowv = jnp.full((_L,), c, jnp.int32)
                    vals = plsc.load_gather(xbig, [rowv, col_l])
                    plsc.store_scatter(rot, [rowv, col_s], vals)

        plsc.parallel_loop(0, _W, 16, unroll=4)(body)
        t0 = pl.multiple_of(q * 4096, 128)
        for half in range(2):
            dst_col = pl.multiple_of(t0 + half * _T, 128)
            pltpu.async_copy(
                rot, out_hbm.at[batch, pl.ds(o8, 8), pl.ds(dst_col, _W)],
                semos[slot])

    def step(s, carry):
        for e in range(2):
            batch = 2 * s + e

            @pl.when(s > 0)
            def _():
                drain_out(e)

            produce(batch, e)
        return carry

    lax.fori_loop(0, _B // 2, step, 0)
    drain_out(0)
    drain_out(1)


_CALL = pl.kernel(
    _jitter_sc,
    out_type=jax.ShapeDtypeStruct((_B, _C, _OUT_T), jnp.float32),
    mesh=_MESH,
    scratch_types=[
        pltpu.VMEM((8, _XW), jnp.float32),
        pltpu.VMEM((8, _W), jnp.float32),
        pltpu.VMEM((8, _W), jnp.float32),
        pltpu.SemaphoreType.DMA,
        pltpu.SemaphoreType.DMA,
        pltpu.SemaphoreType.DMA,
    ],
    compiler_params=pltpu.CompilerParams(needs_layout_passes=False),
)


def kernel(x):
    return _CALL(x)
